# baseline (speedup 1.0000x reference)
import functools

import jax
import jax.numpy as jnp
from jax import lax
from jax.experimental import pallas as pl
from jax.experimental.pallas import tpu as pltpu
from jax.experimental.pallas import tpu_sc as plsc

EMB_DIM = 32
TS = 4
BS = 128
PAD = 129


CV = 800


@functools.partial(jax.jit, static_argnames=("d",))
def _sc_table_transpose(table_t, *, d):
    info = plsc.get_sparse_core_info()
    nc, ns = info.num_cores, info.num_subcores
    nw = nc * ns
    v = table_t.shape[1]
    n_chunks = v // CV
    iters = -(-n_chunks // nw)
    mesh = plsc.VectorSubcoreMesh(core_axis_name="c", subcore_axis_name="s")

    @functools.partial(
        pl.kernel,
        mesh=mesh,
        out_type=jax.ShapeDtypeStruct((v, d), jnp.float32),
        compiler_params=pltpu.CompilerParams(
            use_tc_tiling_on_sc=False, needs_layout_passes=False),
        scratch_types=[
            pltpu.VMEM((2, d, CV + 1), jnp.float32),
            pltpu.VMEM((2, CV, d), jnp.float32),
            pltpu.SemaphoreType.DMA,
            pltpu.SemaphoreType.DMA,
            pltpu.SemaphoreType.DMA,
            pltpu.SemaphoreType.DMA,
        ],
    )
    def k(tt_hbm, out_hbm, in_v, tr_v, si0, si1, so0, so1):
        si = (si0, si1)
        so = (so0, so1)
        wid = lax.axis_index("s") * nc + lax.axis_index("c")
        lane = lax.iota(jnp.int32, 16)

        def chunk_of(g):
            return g * nw + wid

        def start_in(g, b):
            off = chunk_of(g) * CV
            for e in range(d):
                pltpu.async_copy(
                    tt_hbm.at[e, pl.ds(off, CV)],
                    in_v.at[b, e, pl.ds(0, CV)], si[b])

        def wait_in(b):
            for e in range(d):
                pltpu.make_async_copy(
                    tt_hbm.at[e, pl.ds(0, CV)],
                    in_v.at[b, e, pl.ds(0, CV)], si[b]).wait()

        def start_out(g, b):
            pltpu.async_copy(
                tr_v.at[b], out_hbm.at[pl.ds(chunk_of(g) * CV, CV)], so[b])

        def wait_out(b):
            pltpu.make_async_copy(
                tr_v.at[b], out_hbm.at[pl.ds(0, CV)], so[b]).wait()

        e_lo = lane
        e_hi = lane + 16

        def transpose(b):
            def body(u, carry):
                for q in range(2):
                    vv = u * 2 + q
                    v_vec = jnp.full((16,), vv, jnp.int32)
                    lo = plsc.load_gather(in_v.at[b], [e_lo, v_vec])
                    hi = plsc.load_gather(in_v.at[b], [e_hi, v_vec])
                    tr_v[b, vv, pl.ds(0, 16)] = lo
                    tr_v[b, vv, pl.ds(16, 16)] = hi
                return carry

            lax.fori_loop(0, CV // 2, body, 0)

        def active(g):
            return chunk_of(g) < n_chunks

        @pl.when(active(0))
        def _():
            start_in(0, 0)

        @pl.when(active(1))
        def _():
            start_in(1, 1)

        def step(g, b):
            @pl.when(active(g))
            def _():
                wait_in(b)

                @pl.when(g >= 2)
                def _():
                    wait_out(b)

                transpose(b)

                @pl.when(active(g + 2))
                def _():
                    start_in(g + 2, b)

                start_out(g, b)

        def pair(p, carry):
            step(2 * p, 0)
            step(2 * p + 1, 1)
            return carry

        lax.fori_loop(0, iters // 2, pair, 0)

        wait_out(0)
        wait_out(1)

    return k(table_t)


@functools.partial(jax.jit, static_argnames=("d",))
def _sc_gather(table, x4, *, d):
    info = plsc.get_sparse_core_info()
    nc, ns = info.num_cores, info.num_subcores
    nw = nc * ns
    ntt, nbt = x4.shape[0], x4.shape[1]
    t_total = ntt * x4.shape[2]
    bt_per_w = nbt // nw
    et = d // 8
    halves = x4.shape[2] // TS
    n_chunks = bt_per_w * ntt * halves
    mesh = plsc.VectorSubcoreMesh(core_axis_name="c", subcore_axis_name="s")

    @functools.partial(
        pl.kernel,
        mesh=mesh,
        out_type=jax.ShapeDtypeStruct((t_total, et, nbt, 8, 128), jnp.float32),
        compiler_params=pltpu.CompilerParams(
            use_tc_tiling_on_sc=False, needs_layout_passes=False),
        scratch_types=[
            pltpu.VMEM((2, TS, BS), jnp.int32),
            pltpu.VMEM((2, TS, BS, d), jnp.float32),
            pltpu.VMEM((2, TS, et, 8, PAD), jnp.float32),
            pltpu.SemaphoreType.DMA,
            pltpu.SemaphoreType.DMA,
            pltpu.SemaphoreType.DMA,
            pltpu.SemaphoreType.DMA,
            pltpu.SemaphoreType.DMA,
            pltpu.SemaphoreType.DMA,
        ],
    )
    def k(table_hbm, x4_hbm, out_hbm, idx_v, rows_v, trans_v,
          si0, si1, sg0, sg1, so0, so1):
        si = (si0, si1)
        sg = (sg0, sg1)
        so = (so0, so1)
        wid = lax.axis_index("s") * nc + lax.axis_index("c")
        bt0 = wid * bt_per_w

        def coords(g):
            btl = g % bt_per_w
            h = (g // bt_per_w) % halves
            tt = g // (bt_per_w * halves)
            return tt, h, bt0 + btl

        def start_idx(g, b):
            tt, h, bt = coords(g)
            pltpu.async_copy(
                x4_hbm.at[tt, bt, pl.ds(h * TS, TS)], idx_v.at[b], si[b])

        def wait_idx(b):
            pltpu.make_async_copy(
                x4_hbm.at[0, 0, pl.ds(0, TS)], idx_v.at[b], si[b]).wait()

        def start_gather(b):
            for ts in range(TS):
                pltpu.async_copy(
                    table_hbm.at[idx_v.at[b, ts]], rows_v.at[b, ts], sg[b])

        def wait_gather(b):
            for ts in range(TS):
                pltpu.make_async_copy(
                    table_hbm.at[idx_v.at[b, ts]], rows_v.at[b, ts],
                    sg[b]).wait()

        def start_out_ts(g, b, ts):
            tt, h, bt = coords(g)
            t0 = tt * (TS * halves) + h * TS
            for e in range(et):
                pltpu.async_copy(
                    trans_v.at[b, ts, e, pl.ds(0, 8), pl.ds(0, 128)],
                    out_hbm.at[t0 + ts, e, bt], so[b])

        def wait_out(b):
            for ts in range(TS):
                for e in range(et):
                    pltpu.make_async_copy(
                        trans_v.at[b, ts, e, pl.ds(0, 8), pl.ds(0, 128)],
                        out_hbm.at[0, e, 0], so[b]).wait()

        lane = lax.iota(jnp.int32, 16)
        et_lo, es_lo = lane // 8, lane % 8
        et_hi = et_lo + 2

        def transpose_and_out(g, b):
            for ts in range(TS):
                def body(u, bs_vec, ts=ts):
                    for q in range(4):
                        bs = u * 4 + q
                        lo = rows_v[b, ts, bs, pl.ds(0, 16)]
                        hi = rows_v[b, ts, bs, pl.ds(16, 16)]
                        bsv = bs_vec + q
                        plsc.store_scatter(
                            trans_v.at[b, ts], [et_lo, es_lo, bsv], lo)
                        plsc.store_scatter(
                            trans_v.at[b, ts], [et_hi, es_lo, bsv], hi)
                    return bs_vec + 4

                lax.fori_loop(0, BS // 4, body, jnp.zeros((16,), jnp.int32))
                start_out_ts(g, b, ts)

        start_idx(0, 0)
        start_idx(1, 1)
        wait_idx(0)
        start_gather(0)

        def chunk_step(g, b):
            bo = 1 - b
            wait_gather(b)

            @pl.when(g + 2 < n_chunks)
            def _():
                start_idx(g + 2, b)

            @pl.when(g + 1 < n_chunks)
            def _():
                wait_idx(bo)
                start_gather(bo)

            @pl.when(g >= 2)
            def _():
                wait_out(b)

            transpose_and_out(g, b)

        def pair(p, carry):
            chunk_step(2 * p, 0)
            chunk_step(2 * p + 1, 1)
            return carry

        lax.fori_loop(0, n_chunks // 2, pair, 0)

        wait_out(0)
        wait_out(1)

    return k(table, x4)


def kernel(x, table):
    bsz, t = x.shape
    x4 = (x.astype(jnp.int32)
          .T.reshape(t // 8, 8, bsz // 128, 128)
          .transpose(0, 2, 1, 3))
    table_rows = _sc_table_transpose(table.T, d=EMB_DIM)
    out5 = _sc_gather(table_rows, x4, d=EMB_DIM)
    return out5.transpose(2, 4, 0, 1, 3).reshape(bsz, t, EMB_DIM)


# speedup vs baseline: 3.3060x; 3.3060x over previous
import functools

import jax
import jax.numpy as jnp
from jax import lax
from jax.experimental import pallas as pl
from jax.experimental.pallas import tpu as pltpu
from jax.experimental.pallas import tpu_sc as plsc

EMB_DIM = 32
TS = 4
BS = 128
PAD = 129


@functools.partial(jax.jit, static_argnames=("d",))
def _sc_gather(table, x4, *, d):
    info = plsc.get_sparse_core_info()
    nc, ns = info.num_cores, info.num_subcores
    nw = nc * ns
    ntt, nbt = x4.shape[0], x4.shape[1]
    t_total = ntt * x4.shape[2]
    bt_per_w = nbt // nw
    et = d // 8
    halves = x4.shape[2] // TS
    n_chunks = bt_per_w * ntt * halves
    mesh = plsc.VectorSubcoreMesh(core_axis_name="c", subcore_axis_name="s")

    @functools.partial(
        pl.kernel,
        mesh=mesh,
        out_type=jax.ShapeDtypeStruct((t_total, et, nbt, 8, 128), jnp.float32),
        compiler_params=pltpu.CompilerParams(
            use_tc_tiling_on_sc=False, needs_layout_passes=False),
        scratch_types=[
            pltpu.VMEM((2, TS, BS), jnp.int32),
            pltpu.VMEM((2, TS, BS, d), jnp.float32),
            pltpu.VMEM((2, TS, et, 8, PAD), jnp.float32),
            pltpu.SemaphoreType.DMA,
            pltpu.SemaphoreType.DMA,
            pltpu.SemaphoreType.DMA,
            pltpu.SemaphoreType.DMA,
            pltpu.SemaphoreType.DMA,
            pltpu.SemaphoreType.DMA,
        ],
    )
    def k(table_hbm, x4_hbm, out_hbm, idx_v, rows_v, trans_v,
          si0, si1, sg0, sg1, so0, so1):
        si = (si0, si1)
        sg = (sg0, sg1)
        so = (so0, so1)
        wid = lax.axis_index("s") * nc + lax.axis_index("c")
        bt0 = wid * bt_per_w

        def coords(g):
            btl = g % bt_per_w
            h = (g // bt_per_w) % halves
            tt = g // (bt_per_w * halves)
            return tt, h, bt0 + btl

        def start_idx(g, b):
            tt, h, bt = coords(g)
            pltpu.async_copy(
                x4_hbm.at[tt, bt, pl.ds(h * TS, TS)], idx_v.at[b], si[b])

        def wait_idx(b):
            pltpu.make_async_copy(
                x4_hbm.at[0, 0, pl.ds(0, TS)], idx_v.at[b], si[b]).wait()

        def start_gather(b):
            for ts in range(TS):
                pltpu.async_copy(
                    table_hbm.at[idx_v.at[b, ts]], rows_v.at[b, ts], sg[b])

        def wait_gather(b):
            for ts in range(TS):
                pltpu.make_async_copy(
                    table_hbm.at[idx_v.at[b, ts]], rows_v.at[b, ts],
                    sg[b]).wait()

        def start_out_ts(g, b, ts):
            tt, h, bt = coords(g)
            t0 = tt * (TS * halves) + h * TS
            for e in range(et):
                pltpu.async_copy(
                    trans_v.at[b, ts, e, pl.ds(0, 8), pl.ds(0, 128)],
                    out_hbm.at[t0 + ts, e, bt], so[b])

        def wait_out(b):
            for ts in range(TS):
                for e in range(et):
                    pltpu.make_async_copy(
                        trans_v.at[b, ts, e, pl.ds(0, 8), pl.ds(0, 128)],
                        out_hbm.at[0, e, 0], so[b]).wait()

        lane = lax.iota(jnp.int32, 16)
        et_lo, es_lo = lane // 8, lane % 8
        et_hi = et_lo + 2

        def transpose_and_out(g, b):
            for ts in range(TS):
                def body(u, bs_vec, ts=ts):
                    for q in range(4):
                        bs = u * 4 + q
                        lo = rows_v[b, ts, bs, pl.ds(0, 16)]
                        hi = rows_v[b, ts, bs, pl.ds(16, 16)]
                        bsv = bs_vec + q
                        plsc.store_scatter(
                            trans_v.at[b, ts], [et_lo, es_lo, bsv], lo)
                        plsc.store_scatter(
                            trans_v.at[b, ts], [et_hi, es_lo, bsv], hi)
                    return bs_vec + 4

                lax.fori_loop(0, BS // 4, body, jnp.zeros((16,), jnp.int32))
                start_out_ts(g, b, ts)

        start_idx(0, 0)
        start_idx(1, 1)
        wait_idx(0)
        start_gather(0)

        def chunk_step(g, b):
            bo = 1 - b
            wait_gather(b)

            @pl.when(g + 2 < n_chunks)
            def _():
                start_idx(g + 2, b)

            @pl.when(g + 1 < n_chunks)
            def _():
                wait_idx(bo)
                start_gather(bo)

            @pl.when(g >= 2)
            def _():
                wait_out(b)

            transpose_and_out(g, b)

        def pair(p, carry):
            chunk_step(2 * p, 0)
            chunk_step(2 * p + 1, 1)
            return carry

        lax.fori_loop(0, n_chunks // 2, pair, 0)

        wait_out(0)
        wait_out(1)

    return k(table, x4)


def kernel(x, table):
    bsz, t = x.shape
    x4 = (x.astype(jnp.int32)
          .T.reshape(t // 8, 8, bsz // 128, 128)
          .transpose(0, 2, 1, 3))
    out5 = _sc_gather(table, x4, d=EMB_DIM)
    return out5.transpose(2, 4, 0, 1, 3).reshape(bsz, t, EMB_DIM)
